# baseline (speedup 1.0000x reference)
import functools

import jax
import jax.numpy as jnp
from jax import lax
from jax.experimental import pallas as pl
from jax.experimental.pallas import tpu as pltpu
from jax.experimental.pallas import tpu_sc as plsc

_NC = 2
_NS = 16
_NW = _NC * _NS

_B = 4096
_L = 200
_D = 64
_N = _B * _L
_GROUP = 128
_NGROups_TOTAL = _N // _GROUP
_G_PER_W = _NGROups_TOTAL // _NW
_STEPS = _G_PER_W // 2


def _sc_body(x_hbm, tok_hbm, pos_hbm, out_hbm,
             idx_v, pos_v, buf0, buf1, gsem, osem0, osem1):
    wid = lax.axis_index("s") * _NC + lax.axis_index("c")
    g0 = wid * _G_PER_W

    pltpu.sync_copy(x_hbm.at[pl.ds(g0, _G_PER_W)], idx_v)
    pltpu.sync_copy(pos_hbm, pos_v)

    bufs = (buf0, buf1)
    osems = (osem0, osem1)

    def add_pos(buf, off):
        def jbody(j, _):
            r = j * 4
            for u in range(4):
                row = r + u
                p = off + row
                for k in range(4):
                    sl = pl.ds(k * 16, 16)
                    plsc.addupdate(buf.at[row, sl], pos_v[p, sl])
            return _
        lax.fori_loop(0, _GROUP // 4, jbody, 0, unroll=False)

    def step(s, _):
        for b in range(2):
            g = s * 2 + b
            buf = bufs[b]
            @pl.when(s > 0)
            def _wait_prev():
                pltpu.make_async_copy(buf, out_hbm.at[g0], osems[b]).wait()
            pltpu.async_copy(tok_hbm.at[idx_v.at[g]], buf, gsem).wait()
            off = lax.rem(g * _GROUP, _L)
            add_pos(buf, off)
            pltpu.async_copy(buf, out_hbm.at[g0 + g], osems[b])
        return _

    lax.fori_loop(0, _STEPS, step, 0, unroll=False)
    for b in range(2):
        pltpu.make_async_copy(bufs[b], out_hbm.at[g0], osems[b]).wait()


@jax.jit
def _tok_pos_embed(x2, token_table, pos_ext):
    kfn = functools.partial(
        pl.kernel,
        out_type=jax.ShapeDtypeStruct((_NGROups_TOTAL, _GROUP, _D), jnp.float32),
        mesh=plsc.VectorSubcoreMesh(core_axis_name="c", subcore_axis_name="s"),
        scratch_types=[
            pltpu.VMEM((_G_PER_W, _GROUP), jnp.int32),
            pltpu.VMEM((_L + _GROUP, _D), jnp.float32),
            pltpu.VMEM((_GROUP, _D), jnp.float32),
            pltpu.VMEM((_GROUP, _D), jnp.float32),
            pltpu.SemaphoreType.DMA,
            pltpu.SemaphoreType.DMA,
            pltpu.SemaphoreType.DMA,
        ],
        compiler_params=pltpu.CompilerParams(use_tc_tiling_on_sc=False),
    )(_sc_body)
    return kfn(x2, token_table, pos_ext)


def kernel(x, token_table, pos_table):
    x2 = x.astype(jnp.int32).reshape(_NGROups_TOTAL, _GROUP)
    pos_ext = jnp.concatenate([pos_table, pos_table[:_GROUP]], axis=0)
    out = _tok_pos_embed(x2, token_table, pos_ext)
    return out.reshape(_B, _L, _D)


# speedup vs baseline: 1.2268x; 1.2268x over previous
import functools

import jax
import jax.numpy as jnp
from jax import lax
from jax.experimental import pallas as pl
from jax.experimental.pallas import tpu as pltpu
from jax.experimental.pallas import tpu_sc as plsc

_NC = 2
_NS = 16
_NW = _NC * _NS

_B = 4096
_L = 200
_D = 64
_BLK = _B // _NW


def _sc_body(xt_hbm, tok_hbm, pos_hbm, out_hbm,
             idx_v, pos_v, gbuf0, gbuf1, tbuf,
             gsem0, gsem1, osem0, osem1):
    wid = lax.axis_index("s") * _NC + lax.axis_index("c")
    b0 = wid * _BLK

    pltpu.sync_copy(xt_hbm.at[:, pl.ds(b0, _BLK)], idx_v)
    pltpu.sync_copy(pos_hbm, pos_v)

    gbufs = (gbuf0, gbuf1)
    gsems = (gsem0, gsem1)
    osems = (osem0, osem1)

    lanes = lax.iota(jnp.int32, 16)
    svecs = [(16 * k + lanes) >> 3 for k in range(4)]
    d8vecs = [(16 * k + lanes) & 7 for k in range(4)]

    def substep(l, u):
        @pl.when(l + 1 < _L)
        def _prefetch():
            pltpu.async_copy(tok_hbm.at[idx_v.at[l + 1]], gbufs[u ^ 1],
                             gsems[u ^ 1])
        pv = [pos_v[l, pl.ds(16 * k, 16)] for k in range(4)]
        @pl.when(l >= 2)
        def _reclaim():
            pltpu.make_async_copy(tbuf.at[u], out_hbm.at[0, :, wid],
                                  osems[u]).wait()
        pltpu.make_async_copy(tok_hbm.at[idx_v.at[l]], gbufs[u],
                              gsems[u]).wait()
        dst = tbuf.at[u]
        gb = gbufs[u]

        @plsc.parallel_loop(0, _BLK, 1, unroll=8)
        def _rows(b):
            bvec = jnp.full((16,), b, dtype=jnp.int32)
            for k in range(4):
                v = gb[b, pl.ds(16 * k, 16)] + pv[k]
                plsc.store_scatter(dst, [svecs[k], d8vecs[k], bvec], v)
        pltpu.async_copy(tbuf.at[u], out_hbm.at[l, :, wid], osems[u])

    pltpu.async_copy(tok_hbm.at[idx_v.at[0]], gbuf0, gsem0)

    def step(i, _):
        substep(2 * i, 0)
        substep(2 * i + 1, 1)
        return _

    lax.fori_loop(0, _L // 2, step, 0, unroll=False)
    for u in range(2):
        pltpu.make_async_copy(tbuf.at[u], out_hbm.at[0, :, wid],
                              osems[u]).wait()


@jax.jit
def _tok_pos_embed(xt, token_table, pos_table):
    kfn = functools.partial(
        pl.kernel,
        out_type=jax.ShapeDtypeStruct((_L, 8, _NW, 8, _BLK), jnp.float32),
        mesh=plsc.VectorSubcoreMesh(core_axis_name="c", subcore_axis_name="s"),
        scratch_types=[
            pltpu.VMEM((_L, _BLK), jnp.int32),
            pltpu.VMEM((_L, _D), jnp.float32),
            pltpu.VMEM((_BLK, _D), jnp.float32),
            pltpu.VMEM((_BLK, _D), jnp.float32),
            pltpu.VMEM((2, 8, 8, _BLK), jnp.float32),
            pltpu.SemaphoreType.DMA,
            pltpu.SemaphoreType.DMA,
            pltpu.SemaphoreType.DMA,
            pltpu.SemaphoreType.DMA,
        ],
        compiler_params=pltpu.CompilerParams(use_tc_tiling_on_sc=False,
                                             needs_layout_passes=False),
    )(_sc_body)
    return kfn(xt, token_table, pos_table)


def kernel(x, token_table, pos_table):
    xt = x.astype(jnp.int32).T
    out5 = _tok_pos_embed(xt, token_table, pos_table)
    return out5.transpose(2, 4, 0, 1, 3).reshape(_B, _L, _D)


# speedup vs baseline: 1.2458x; 1.0155x over previous
import functools

import numpy as np

import jax
import jax.numpy as jnp
from jax import lax
from jax.experimental import pallas as pl
from jax.experimental.pallas import tpu as pltpu
from jax.experimental.pallas import tpu_sc as plsc

_NC = 2
_NS = 16
_NW = _NC * _NS

_B = 4096
_L = 200
_D = 64
_BLK = _B // _NW
_NBUF = 4


def _sc_body(xt_hbm, tok_hbm, pos_hbm, out_hbm,
             idx_v, pos_v, gbuf0, gbuf1, gbuf2, gbuf3, tbuf,
             gsem0, gsem1, gsem2, gsem3, osem0, osem1):
    wid = lax.axis_index("s") * _NC + lax.axis_index("c")
    b0 = wid * _BLK

    pltpu.sync_copy(xt_hbm.at[:, pl.ds(b0, _BLK)], idx_v)
    pltpu.sync_copy(pos_hbm, pos_v)

    gbufs = (gbuf0, gbuf1, gbuf2, gbuf3)
    gsems = (gsem0, gsem1, gsem2, gsem3)
    osems = (osem0, osem1)

    lanes = lax.iota(jnp.int32, 16)
    svecs = [(16 * k + lanes) >> 3 for k in range(4)]
    d8vecs = [(16 * k + lanes) & 7 for k in range(4)]

    def fire_gather(l, slot):
        @pl.when(l < _L)
        def _():
            pltpu.async_copy(tok_hbm.at[idx_v.at[l]], gbufs[slot],
                             gsems[slot])

    def substep(l, u):
        fire_gather(l + _NBUF - 1, (u + _NBUF - 1) % _NBUF)
        pv = [pos_v[l, pl.ds(16 * k, 16)] for k in range(4)]
        @pl.when(l >= 2)
        def _reclaim():
            pltpu.make_async_copy(tbuf.at[u % 2], out_hbm.at[0, :, wid],
                                  osems[u % 2]).wait()
        pltpu.make_async_copy(tok_hbm.at[idx_v.at[l]], gbufs[u],
                              gsems[u]).wait()
        dst = tbuf.at[u % 2]
        gb = gbufs[u]

        @plsc.parallel_loop(0, _BLK, 1, unroll=8)
        def _rows(b):
            bvec = jnp.full((16,), b, dtype=jnp.int32)
            for k in range(4):
                v = gb[b, pl.ds(16 * k, 16)] + pv[k]
                plsc.store_scatter(dst, [svecs[k], d8vecs[k], bvec], v)

        pltpu.async_copy(tbuf.at[u % 2], out_hbm.at[l, :, wid], osems[u % 2])

    for l in range(_NBUF - 1):
        fire_gather(l, l)

    def step(i, _):
        for u in range(_NBUF):
            substep(_NBUF * i + u, u)
        return _

    lax.fori_loop(0, _L // _NBUF, step, 0, unroll=False)
    for u in range(2):
        pltpu.make_async_copy(tbuf.at[u], out_hbm.at[0, :, wid],
                              osems[u]).wait()


@jax.jit
def _tok_pos_embed(xt, token_table, pos_table):
    kfn = functools.partial(
        pl.kernel,
        out_type=jax.ShapeDtypeStruct((_L, 8, _NW, 8, _BLK), jnp.float32),
        mesh=plsc.VectorSubcoreMesh(core_axis_name="c", subcore_axis_name="s"),
        scratch_types=[
            pltpu.VMEM((_L, _BLK), jnp.int32),
            pltpu.VMEM((_L, _D), jnp.float32),
            pltpu.VMEM((_BLK, _D), jnp.float32),
            pltpu.VMEM((_BLK, _D), jnp.float32),
            pltpu.VMEM((_BLK, _D), jnp.float32),
            pltpu.VMEM((_BLK, _D), jnp.float32),
            pltpu.VMEM((2, 8, 8, _BLK), jnp.float32),
            pltpu.SemaphoreType.DMA,
            pltpu.SemaphoreType.DMA,
            pltpu.SemaphoreType.DMA,
            pltpu.SemaphoreType.DMA,
            pltpu.SemaphoreType.DMA,
            pltpu.SemaphoreType.DMA,
        ],
        compiler_params=pltpu.CompilerParams(use_tc_tiling_on_sc=False,
                                             needs_layout_passes=False),
    )(_sc_body)
    return kfn(xt, token_table, pos_table)


def kernel(x, token_table, pos_table):
    xt = x.astype(jnp.int32).T
    out5 = _tok_pos_embed(xt, token_table, pos_table)
    return out5.transpose(2, 4, 0, 1, 3).reshape(_B, _L, _D)


# speedup vs baseline: 2.2112x; 1.7749x over previous
import functools

import numpy as np

import jax
import jax.numpy as jnp
from jax import lax
from jax.experimental import pallas as pl
from jax.experimental.pallas import tpu as pltpu
from jax.experimental.pallas import tpu_sc as plsc

_NC = 2
_NS = 16
_NW = _NC * _NS

_B = 4096
_L = 200
_D = 64
_BLK = _B // _NW
_NBUF = 4


def _sc_body(xt_hbm, tok_hbm, pos_hbm, out_hbm,
             idx_v, pos_v, gbuf0, gbuf1, gbuf2, gbuf3, tbuf,
             gsem0, gsem1, gsem2, gsem3, osem0, osem1):
    wid = lax.axis_index("s") * _NC + lax.axis_index("c")
    b0 = wid * _BLK

    pltpu.sync_copy(xt_hbm.at[:, pl.ds(b0, _BLK)], idx_v)
    pltpu.sync_copy(pos_hbm, pos_v)

    gbufs = (gbuf0, gbuf1, gbuf2, gbuf3)
    gsems = (gsem0, gsem1, gsem2, gsem3)
    osems = (osem0, osem1)

    lanes = lax.iota(jnp.int32, 16)
    dvecs = [16 * k + lanes for k in range(4)]

    def fire_gather(l, slot):
        @pl.when(l < _L)
        def _():
            pltpu.async_copy(tok_hbm.at[idx_v.at[l]], gbufs[slot],
                             gsems[slot])

    def substep(l, u):
        fire_gather(l + _NBUF - 1, (u + _NBUF - 1) % _NBUF)
        pv = [pos_v[l, pl.ds(16 * k, 16)] for k in range(4)]
        @pl.when(l >= 2)
        def _reclaim():
            for s in range(8):
                pltpu.make_async_copy(
                    tbuf.at[u % 2, pl.ds(8 * s, 8), pl.ds(0, _BLK)],
                    out_hbm.at[0, s, wid], osems[u % 2]).wait()
        pltpu.make_async_copy(tok_hbm.at[idx_v.at[l]], gbufs[u],
                              gsems[u]).wait()
        dst = tbuf.at[u % 2]
        gb = gbufs[u]

        @plsc.parallel_loop(0, _BLK, 1, unroll=8)
        def _rows(b):
            bvec = jnp.full((16,), b, dtype=jnp.int32)
            for k in range(4):
                v = gb[b, pl.ds(16 * k, 16)] + pv[k]
                plsc.store_scatter(dst, [dvecs[k], bvec], v)

        for s in range(8):
            pltpu.async_copy(
                tbuf.at[u % 2, pl.ds(8 * s, 8), pl.ds(0, _BLK)],
                out_hbm.at[l, s, wid], osems[u % 2])

    for l in range(_NBUF - 1):
        fire_gather(l, l)

    def step(i, _):
        for u in range(_NBUF):
            substep(_NBUF * i + u, u)
        return _

    lax.fori_loop(0, _L // _NBUF, step, 0, unroll=False)
    for u in range(2):
        for s in range(8):
            pltpu.make_async_copy(
                tbuf.at[u, pl.ds(8 * s, 8), pl.ds(0, _BLK)],
                out_hbm.at[0, s, wid], osems[u]).wait()


@jax.jit
def _tok_pos_embed(xt, token_table, pos_table):
    kfn = functools.partial(
        pl.kernel,
        out_type=jax.ShapeDtypeStruct((_L, 8, _NW, 8, _BLK), jnp.float32),
        mesh=plsc.VectorSubcoreMesh(core_axis_name="c", subcore_axis_name="s"),
        scratch_types=[
            pltpu.VMEM((_L, _BLK), jnp.int32),
            pltpu.VMEM((_L, _D), jnp.float32),
            pltpu.VMEM((_BLK, _D), jnp.float32),
            pltpu.VMEM((_BLK, _D), jnp.float32),
            pltpu.VMEM((_BLK, _D), jnp.float32),
            pltpu.VMEM((_BLK, _D), jnp.float32),
            pltpu.VMEM((2, _D, 129), jnp.float32),
            pltpu.SemaphoreType.DMA,
            pltpu.SemaphoreType.DMA,
            pltpu.SemaphoreType.DMA,
            pltpu.SemaphoreType.DMA,
            pltpu.SemaphoreType.DMA,
            pltpu.SemaphoreType.DMA,
        ],
        compiler_params=pltpu.CompilerParams(use_tc_tiling_on_sc=False,
                                             needs_layout_passes=False),
    )(_sc_body)
    return kfn(xt, token_table, pos_table)


def kernel(x, token_table, pos_table):
    xt = x.astype(jnp.int32).T
    out5 = _tok_pos_embed(xt, token_table, pos_table)
    return out5.transpose(2, 4, 0, 1, 3).reshape(_B, _L, _D)
